# Initial kernel scaffold; baseline (speedup 1.0000x reference)
import jax
import jax.numpy as jnp
from jax.experimental import pallas as pl


def kernel(inputs, embedding_table):
    raise NotImplementedError("write your pallas kernel here")


# speedup vs baseline: 1.6244x; 1.6244x over previous
import functools

import jax
import jax.numpy as jnp
from jax import lax
from jax.experimental import pallas as pl
from jax.experimental.pallas import tpu as pltpu
from jax.experimental.pallas import tpu_sc as plsc

VOCAB_ROWS = 32
EMBED_DIM = 64
BATCH = 4096
SEQ = 200
TOTAL = BATCH * SEQ

_info = plsc.get_sparse_core_info()
_NC = _info.num_cores
_NS = _info.num_subcores
_NW = _NC * _NS
PER_W = TOTAL // _NW
CHUNK = 128
N_CHUNKS = PER_W // CHUNK


def _make_kernel():
    mesh = plsc.VectorSubcoreMesh(core_axis_name="c", subcore_axis_name="s")

    @functools.partial(
        pl.kernel,
        mesh=mesh,
        out_type=jax.ShapeDtypeStruct((TOTAL, EMBED_DIM), jnp.float32),
        compiler_params=pltpu.CompilerParams(use_tc_tiling_on_sc=False),
        scratch_types=[
            pltpu.VMEM((PER_W,), jnp.int32),
            pltpu.VMEM((CHUNK, EMBED_DIM), jnp.float32),
            pltpu.SemaphoreType.DMA,
        ],
    )
    def k(idx_hbm, table_hbm, out_hbm, idx_v, rows_v, sem):
        wid = lax.axis_index("s") * _NC + lax.axis_index("c")
        base = wid * PER_W
        pltpu.sync_copy(idx_hbm.at[pl.ds(base, PER_W)], idx_v)

        def body(g, carry):
            off = g * CHUNK
            pltpu.async_copy(
                table_hbm.at[idx_v.at[pl.ds(off, CHUNK)]], rows_v, sem
            ).wait()
            pltpu.sync_copy(rows_v, out_hbm.at[pl.ds(base + off, CHUNK)])
            return carry

        lax.fori_loop(0, N_CHUNKS, body, 0)

    return k


_sc_gather = _make_kernel()


def kernel(inputs, embedding_table):
    idx = inputs.reshape(TOTAL)
    out = _sc_gather(idx, embedding_table)
    return out.reshape(BATCH, SEQ, EMBED_DIM)


# speedup vs baseline: 5.0415x; 3.1035x over previous
import functools

import jax
import jax.numpy as jnp
from jax import lax
from jax.experimental import pallas as pl
from jax.experimental.pallas import tpu as pltpu
from jax.experimental.pallas import tpu_sc as plsc

VOCAB_ROWS = 32
EMBED_DIM = 64
BATCH = 4096
SEQ = 200
TOTAL = BATCH * SEQ

_info = plsc.get_sparse_core_info()
_NC = _info.num_cores
_NS = _info.num_subcores
_NW = _NC * _NS
PER_W = TOTAL // _NW
CHUNK = 128
N_CHUNKS = PER_W // CHUNK
NBUF = 4
SKEW = 2


def _make_kernel():
    mesh = plsc.VectorSubcoreMesh(core_axis_name="c", subcore_axis_name="s")

    @functools.partial(
        pl.kernel,
        mesh=mesh,
        out_type=jax.ShapeDtypeStruct((TOTAL, EMBED_DIM), jnp.float32),
        compiler_params=pltpu.CompilerParams(use_tc_tiling_on_sc=False),
        scratch_types=[
            pltpu.VMEM((N_CHUNKS, CHUNK), jnp.int32),
            pltpu.VMEM((NBUF, CHUNK, EMBED_DIM), jnp.float32),
            pltpu.VMEM_SHARED((VOCAB_ROWS, EMBED_DIM), jnp.float32),
        ]
        + [pltpu.SemaphoreType.DMA] * (2 * NBUF),
    )
    def k(idx_hbm, table_hbm, out_hbm, idx_v, rows, table_sh,
          g0, g1, g2, g3, o0, o1, o2, o3):
        gsem = [g0, g1, g2, g3]
        osem = [o0, o1, o2, o3]
        sid = lax.axis_index("s")
        wid = sid * _NC + lax.axis_index("c")
        base = wid * PER_W

        @pl.when(sid == 0)
        def _():
            pltpu.sync_copy(table_hbm, table_sh)

        plsc.subcore_barrier()

        pltpu.sync_copy(idx_hbm.at[pl.ds(wid * N_CHUNKS, N_CHUNKS)], idx_v)

        def start_gather(g, b):
            return pltpu.async_copy(table_sh.at[idx_v.at[g]], rows.at[b],
                                    gsem[b])

        def gather_wait(g, b):
            pltpu.make_async_copy(table_sh.at[idx_v.at[g]], rows.at[b],
                                  gsem[b]).wait()

        def out_slice(g):
            return out_hbm.at[pl.ds(base + g * CHUNK, CHUNK)]

        def start_write(g, b):
            return pltpu.async_copy(rows.at[b], out_slice(g), osem[b])

        def write_wait(g, b):
            pltpu.make_async_copy(rows.at[b], out_slice(g), osem[b]).wait()

        for b in range(NBUF):
            start_gather(b, b)
        for b in range(SKEW):
            gather_wait(b, b)
            start_write(b, b)

        def body(i, carry):
            gbase = i * NBUF
            for b in range(NBUF):
                g = gbase + b
                write_wait(g - NBUF, b)
                start_gather(g, b)
                gw = g - SKEW
                bw = (b + NBUF - SKEW) % NBUF
                gather_wait(gw, bw)
                start_write(gw, bw)
            return carry

        lax.fori_loop(1, N_CHUNKS // NBUF, body, 0)

        last = N_CHUNKS - NBUF
        for b in range(SKEW, NBUF):
            g = last + b
            gather_wait(g, b)
            start_write(g, b)
        for b in range(NBUF):
            write_wait(last + b, b)

    return k


_sc_gather = _make_kernel()


def kernel(inputs, embedding_table):
    idx = inputs.reshape(TOTAL // CHUNK, CHUNK)
    out = _sc_gather(idx, embedding_table)
    return out.reshape(BATCH, SEQ, EMBED_DIM)
